# Initial kernel scaffold; baseline (speedup 1.0000x reference)
import jax
import jax.numpy as jnp
from jax.experimental import pallas as pl


def kernel(input_seq, token_table, pos_table):
    raise NotImplementedError("write your pallas kernel here")


# speedup vs baseline: 2.2196x; 2.2196x over previous
import jax
import jax.numpy as jnp
from jax import lax
from jax.experimental import pallas as pl
from jax.experimental.pallas import tpu as pltpu
from jax.experimental.pallas import tpu_sc as plsc

VOCAB = 100000
EMBED = 64
MAX_LEN = 200
BATCH = 4096

NUM_CORES = 2
NUM_SUBCORES = 16
NW = NUM_CORES * NUM_SUBCORES
LANES = 16

TOTAL = BATCH * MAX_LEN
PER_W = TOTAL // NW
GROUP = 128
NGROUPS = PER_W // GROUP
D_CHUNKS = EMBED // LANES


def _body(idx_hbm, table_hbm, pos_hbm, out_hbm, idx_v, rows_v, pos_v, sem):
    wid = lax.axis_index("s") * NUM_CORES + lax.axis_index("c")

    pltpu.sync_copy(pos_hbm, pos_v)

    @pl.loop(0, NGROUPS)
    def _group(g):
        pltpu.sync_copy(idx_hbm.at[wid, g], idx_v)
        pltpu.async_copy(table_hbm.at[idx_v], rows_v, sem).wait()

        start = lax.rem(g * GROUP, MAX_LEN)

        @pl.loop(0, GROUP)
        def _row(r):
            for d in range(D_CHUNKS):
                x = pos_v[start + r, pl.ds(d * LANES, LANES)]
                plsc.addupdate(rows_v.at[r, pl.ds(d * LANES, LANES)], x)

        pltpu.sync_copy(rows_v, out_hbm.at[wid, g])


@jax.jit
def _embed(idx, token_table, pos_ext):
    mesh = plsc.VectorSubcoreMesh(
        core_axis_name="c", subcore_axis_name="s",
        num_cores=NUM_CORES, num_subcores=NUM_SUBCORES)
    return pl.kernel(
        _body,
        out_type=jax.ShapeDtypeStruct((NW, NGROUPS, GROUP, EMBED), jnp.float32),
        mesh=mesh,
        compiler_params=pltpu.CompilerParams(use_tc_tiling_on_sc=False),
        scratch_types=[
            pltpu.VMEM((GROUP,), jnp.int32),
            pltpu.VMEM((GROUP, EMBED), jnp.float32),
            pltpu.VMEM((MAX_LEN + GROUP, EMBED), jnp.float32),
            pltpu.SemaphoreType.DMA,
        ],
    )(idx, token_table, pos_ext)


def kernel(input_seq, token_table, pos_table):
    idx = input_seq.astype(jnp.int32).reshape(NW, NGROUPS, GROUP)
    pos_ext = jnp.concatenate([pos_table, pos_table[:GROUP]], axis=0)
    out = _embed(idx, token_table, pos_ext)
    return out.reshape(BATCH, MAX_LEN, EMBED)


# speedup vs baseline: 2.9028x; 1.3078x over previous
import jax
import jax.numpy as jnp
from jax import lax
from jax.experimental import pallas as pl
from jax.experimental.pallas import tpu as pltpu
from jax.experimental.pallas import tpu_sc as plsc

VOCAB = 100000
EMBED = 64
MAX_LEN = 200
BATCH = 4096

NUM_CORES = 2
NUM_SUBCORES = 16
NW = NUM_CORES * NUM_SUBCORES
LANES = 16

TOTAL = BATCH * MAX_LEN
PER_W = TOTAL // NW
GROUP = 128
NGROUPS = PER_W // GROUP
D_CHUNKS = EMBED // LANES
NBUF = 8
NROUNDS = NGROUPS // NBUF


def _body(idx_hbm, table_hbm, pos_hbm, out_hbm, idx_v, pos_v, rows, gsem, osem):
    wid = lax.axis_index("s") * NUM_CORES + lax.axis_index("c")

    pltpu.sync_copy(idx_hbm.at[wid], idx_v)
    pltpu.sync_copy(pos_hbm, pos_v)

    def issue_gather(b, g):
        pltpu.async_copy(table_hbm.at[idx_v.at[g]], rows[b], gsem[b])

    for b in range(NBUF):
        issue_gather(b, b)

    @pl.loop(0, NROUNDS)
    def _round(r):
        base = r * NBUF
        for b in range(NBUF):
            g = base + b
            pltpu.make_async_copy(table_hbm.at[idx_v.at[g]], rows[b], gsem[b]).wait()

            start = lax.rem(g * GROUP, MAX_LEN)

            @pl.loop(0, GROUP, unroll=4)
            def _row(i):
                for d in range(D_CHUNKS):
                    x = pos_v[start + i, pl.ds(d * LANES, LANES)]
                    plsc.addupdate(rows[b].at[i, pl.ds(d * LANES, LANES)], x)

            pltpu.async_copy(rows[b], out_hbm.at[wid, g], osem[b])

        @pl.when(r < NROUNDS - 1)
        def _refill():
            for b in range(NBUF):
                g = base + b
                pltpu.make_async_copy(rows[b], out_hbm.at[wid, g], osem[b]).wait()
                issue_gather(b, g + NBUF)

    for b in range(NBUF):
        g = NGROUPS - NBUF + b
        pltpu.make_async_copy(rows[b], out_hbm.at[wid, g], osem[b]).wait()


@jax.jit
def _embed(idx, token_table, pos_ext):
    mesh = plsc.VectorSubcoreMesh(
        core_axis_name="c", subcore_axis_name="s",
        num_cores=NUM_CORES, num_subcores=NUM_SUBCORES)
    return pl.kernel(
        _body,
        out_type=jax.ShapeDtypeStruct((NW, NGROUPS, GROUP, EMBED), jnp.float32),
        mesh=mesh,
        compiler_params=pltpu.CompilerParams(use_tc_tiling_on_sc=False),
        scratch_types=[
            pltpu.VMEM((NGROUPS, GROUP), jnp.int32),
            pltpu.VMEM((MAX_LEN + GROUP, EMBED), jnp.float32),
            [pltpu.VMEM((GROUP, EMBED), jnp.float32) for _ in range(NBUF)],
            [pltpu.SemaphoreType.DMA for _ in range(NBUF)],
            [pltpu.SemaphoreType.DMA for _ in range(NBUF)],
        ],
    )(idx, token_table, pos_ext)


def kernel(input_seq, token_table, pos_table):
    idx = input_seq.astype(jnp.int32).reshape(NW, NGROUPS, GROUP)
    pos_ext = jnp.concatenate([pos_table, pos_table[:GROUP]], axis=0)
    out = _embed(idx, token_table, pos_ext)
    return out.reshape(BATCH, MAX_LEN, EMBED)


# speedup vs baseline: 3.9551x; 1.3625x over previous
import jax
import jax.numpy as jnp
from jax import lax
from jax.experimental import pallas as pl
from jax.experimental.pallas import tpu as pltpu
from jax.experimental.pallas import tpu_sc as plsc

VOCAB = 100000
EMBED = 64
MAX_LEN = 200
BATCH = 4096

NUM_CORES = 2
NUM_SUBCORES = 16
NW = NUM_CORES * NUM_SUBCORES
LANES = 16

ROWS_PER_W = BATCH // NW
SPLIT = 128
REST = MAX_LEN - SPLIT
D_CHUNKS = EMBED // LANES
NBUF = 4
NROUNDS = ROWS_PER_W // NBUF


def _body(idx_hbm, table_hbm, pos_hbm, out_hbm, idx_v, pos_v, rows, gsem, osem):
    wid = lax.axis_index("s") * NUM_CORES + lax.axis_index("c")
    row0 = wid * ROWS_PER_W

    pltpu.sync_copy(idx_hbm.at[pl.ds(row0, ROWS_PER_W)], idx_v)
    pltpu.sync_copy(pos_hbm, pos_v)

    def issue_gathers(b, r):
        pltpu.async_copy(table_hbm.at[idx_v.at[r, pl.ds(0, SPLIT)]],
                         rows[b].at[pl.ds(0, SPLIT)], gsem[b])
        pltpu.async_copy(table_hbm.at[idx_v.at[r, pl.ds(SPLIT, REST)]],
                         rows[b].at[pl.ds(SPLIT, REST)], gsem[b])

    def wait_gathers(b, r):
        pltpu.make_async_copy(table_hbm.at[idx_v.at[r, pl.ds(0, SPLIT)]],
                              rows[b].at[pl.ds(0, SPLIT)], gsem[b]).wait()
        pltpu.make_async_copy(table_hbm.at[idx_v.at[r, pl.ds(SPLIT, REST)]],
                              rows[b].at[pl.ds(SPLIT, REST)], gsem[b]).wait()

    def wait_out(b, r):
        pltpu.make_async_copy(rows[b], out_hbm.at[row0 + r], osem[b]).wait()

    for b in range(NBUF):
        issue_gathers(b, b)

    @pl.loop(0, NROUNDS)
    def _round(rnd):
        base = rnd * NBUF
        for b in range(NBUF):
            r = base + b
            wait_gathers(b, r)

            @pl.loop(0, MAX_LEN, unroll=4)
            def _pos(i):
                for d in range(D_CHUNKS):
                    x = pos_v[i, pl.ds(d * LANES, LANES)]
                    plsc.addupdate(rows[b].at[i, pl.ds(d * LANES, LANES)], x)

            pltpu.async_copy(rows[b], out_hbm.at[row0 + r], osem[b])

        @pl.when(rnd < NROUNDS - 1)
        def _refill():
            for b in range(NBUF):
                wait_out(b, base + b)
                issue_gathers(b, base + b + NBUF)

    for b in range(NBUF):
        wait_out(b, ROWS_PER_W - NBUF + b)


@jax.jit
def _embed(idx, token_table, pos_table):
    mesh = plsc.VectorSubcoreMesh(
        core_axis_name="c", subcore_axis_name="s",
        num_cores=NUM_CORES, num_subcores=NUM_SUBCORES)
    return pl.kernel(
        _body,
        out_type=jax.ShapeDtypeStruct((BATCH, MAX_LEN, EMBED), jnp.float32),
        mesh=mesh,
        compiler_params=pltpu.CompilerParams(use_tc_tiling_on_sc=False),
        scratch_types=[
            pltpu.VMEM((ROWS_PER_W, MAX_LEN), jnp.int32),
            pltpu.VMEM((MAX_LEN, EMBED), jnp.float32),
            [pltpu.VMEM((MAX_LEN, EMBED), jnp.float32) for _ in range(NBUF)],
            [pltpu.SemaphoreType.DMA for _ in range(NBUF)],
            [pltpu.SemaphoreType.DMA for _ in range(NBUF)],
        ],
    )(idx, token_table, pos_table)


def kernel(input_seq, token_table, pos_table):
    return _embed(input_seq.astype(jnp.int32), token_table, pos_table)


# speedup vs baseline: 3.9674x; 1.0031x over previous
import jax
import jax.numpy as jnp
from jax import lax
from jax.experimental import pallas as pl
from jax.experimental.pallas import tpu as pltpu
from jax.experimental.pallas import tpu_sc as plsc

VOCAB = 100000
EMBED = 64
MAX_LEN = 200
BATCH = 4096

NUM_CORES = 2
NUM_SUBCORES = 16
NW = NUM_CORES * NUM_SUBCORES
LANES = 16

ROWS_PER_W = BATCH // NW
SPLIT = 128
REST = MAX_LEN - SPLIT
NBUF = 4
NOBUF = 2
NROUNDS = ROWS_PER_W // NBUF

HALF = MAX_LEN // 2
MID_ROWS = BATCH * HALF
P_CHUNKS = 128 // LANES


def _body(idx_hbm, table_hbm, pos_hbm, out_hbm, idx_v, pos_v, ins, outs,
          gsem, osem):
    wid = lax.axis_index("s") * NUM_CORES + lax.axis_index("c")
    row0 = wid * ROWS_PER_W

    pltpu.sync_copy(idx_hbm.at[pl.ds(row0, ROWS_PER_W)], idx_v)
    pltpu.sync_copy(pos_hbm, pos_v)

    def gather_copies(b, r):
        return (
            (table_hbm.at[idx_v.at[r, pl.ds(0, SPLIT)]],
             ins[b].at[pl.ds(0, SPLIT)]),
            (table_hbm.at[idx_v.at[r, pl.ds(SPLIT, REST)]],
             ins[b].at[pl.ds(SPLIT, REST)]),
        )

    def issue_gathers(b, r):
        for src, dst in gather_copies(b, r):
            pltpu.async_copy(src, dst, gsem[b])

    def wait_gathers(b, r):
        for src, dst in gather_copies(b, r):
            pltpu.make_async_copy(src, dst, gsem[b]).wait()

    def out_dst(r):
        g = row0 + r
        return out_hbm.at[g // 8, :, lax.rem(g, 8)]

    for b in range(NBUF):
        issue_gathers(b, b)

    @pl.loop(0, NROUNDS)
    def _round(rnd):
        base = rnd * NBUF
        for b in range(NBUF):
            r = base + b
            ob = b % NOBUF
            wait_gathers(b, r)

            @pl.when(r >= NOBUF)
            def _():
                pltpu.make_async_copy(outs[ob], out_dst(r - NOBUF),
                                      osem[ob]).wait()

            @pl.loop(0, HALF, unroll=4)
            def _pos(k):
                for c in range(P_CHUNKS):
                    x = ins[b][2 * k + c // 4, pl.ds(16 * (c % 4), LANES)]
                    p = pos_v[k, pl.ds(c * LANES, LANES)]
                    outs[ob][k, pl.ds(c * LANES, LANES)] = x + p

            pltpu.async_copy(outs[ob], out_dst(r), osem[ob])

            @pl.when(r + NBUF < ROWS_PER_W)
            def _():
                issue_gathers(b, r + NBUF)

    for i in range(NOBUF):
        r = ROWS_PER_W - NOBUF + i
        pltpu.make_async_copy(outs[r % NOBUF], out_dst(r),
                              osem[r % NOBUF]).wait()


@jax.jit
def _embed(idx, token_table, pos2):
    mesh = plsc.VectorSubcoreMesh(
        core_axis_name="c", subcore_axis_name="s",
        num_cores=NUM_CORES, num_subcores=NUM_SUBCORES)
    mid = pl.kernel(
        _body,
        out_type=jax.ShapeDtypeStruct((BATCH // 8, HALF, 8, 128), jnp.float32),
        mesh=mesh,
        compiler_params=pltpu.CompilerParams(use_tc_tiling_on_sc=False),
        scratch_types=[
            pltpu.VMEM((ROWS_PER_W, MAX_LEN), jnp.int32),
            pltpu.VMEM((HALF, 128), jnp.float32),
            [pltpu.VMEM((MAX_LEN, EMBED), jnp.float32) for _ in range(NBUF)],
            [pltpu.VMEM((HALF, 128), jnp.float32) for _ in range(NOBUF)],
            [pltpu.SemaphoreType.DMA for _ in range(NBUF)],
            [pltpu.SemaphoreType.DMA for _ in range(NOBUF)],
        ],
    )(idx, token_table, pos2)
    mid2 = mid.transpose(0, 2, 1, 3).reshape(BATCH, MAX_LEN * EMBED)
    out2 = mid2.T
    return jnp.transpose(out2.reshape(MAX_LEN, EMBED, BATCH), (2, 0, 1))


def kernel(input_seq, token_table, pos_table):
    return _embed(input_seq.astype(jnp.int32), token_table,
                  pos_table.reshape(HALF, 128))
